# baseline (speedup 1.0000x reference)
import jax
import jax.numpy as jnp
from jax import lax
from jax.experimental import pallas as pl

_NAT = 32768
_NSYS = 16
_PREV = int(1.2 * _NAT) + 1
_ADD = _PREV - _NAT


def _pad_kernel(sp_ref, na_ref, bi_ref, co_ref, ce_ref,
                sp_o, na_o, bi_o, co_o, ce_o, ta_o, ts_o):
    sp = sp_ref[:]
    sp_o[: _NAT] = sp
    sp_o[_NAT:] = jnp.full((_ADD,), -1, jnp.int32)

    ta_o[: _NAT] = sp > 0
    ta_o[_NAT:] = jnp.zeros((_ADD,), jnp.bool_)

    bi_o[: _NAT] = bi_ref[:]
    bi_o[_NAT:] = jnp.full((_ADD,), _NSYS, jnp.int32)

    co_o[: _NAT * 3] = co_ref[:]
    co_o[_NAT * 3:] = jnp.zeros((_ADD * 3,), jnp.float32)

    na_o[: _NSYS] = na_ref[:]
    na_o[_NSYS:] = jnp.full((1,), _ADD, jnp.int32)

    ce_o[: _NSYS] = ce_ref[:]
    r = lax.broadcasted_iota(jnp.int32, (1, 3, 3), 1)
    c = lax.broadcasted_iota(jnp.int32, (1, 3, 3), 2)
    ce_o[_NSYS:] = jnp.where(r == c, 1.0, 0.0).astype(jnp.float32)

    ts_o[: _NSYS] = jnp.ones((_NSYS,), jnp.bool_)
    ts_o[_NSYS:] = jnp.zeros((1,), jnp.bool_)


def kernel(species, natoms, batch_index, coordinates, cells):
    coords_flat = coordinates.reshape(-1)
    out = pl.pallas_call(
        _pad_kernel,
        out_shape=(
            jax.ShapeDtypeStruct((_PREV,), jnp.int32),
            jax.ShapeDtypeStruct((_NSYS + 1,), jnp.int32),
            jax.ShapeDtypeStruct((_PREV,), jnp.int32),
            jax.ShapeDtypeStruct((_PREV * 3,), jnp.float32),
            jax.ShapeDtypeStruct((_NSYS + 1, 3, 3), jnp.float32),
            jax.ShapeDtypeStruct((_PREV,), jnp.bool_),
            jax.ShapeDtypeStruct((_NSYS + 1,), jnp.bool_),
        ),
    )(species, natoms, batch_index, coords_flat, cells)
    (species_out, natoms_out, batch_index_out,
     coords_out_flat, cells_out, true_atoms, true_sys) = out
    coordinates_out = coords_out_flat.reshape(_PREV, 3)
    return (species_out, natoms_out, batch_index_out,
            coordinates_out, cells_out, true_atoms, true_sys)


# speedup vs baseline: 7.6736x; 7.6736x over previous
import jax
import jax.numpy as jnp
from jax import lax
from jax.experimental import pallas as pl

_NAT = 32768
_NSYS = 16
_PREV = int(1.2 * _NAT) + 1
_ADD = _PREV - _NAT


def _pad_kernel(sp_ref, na_ref, bi_ref, co_ref, ce_ref,
                sp_o, na_o, bi_o, co_o, ce_o, ta_o, ts_o):
    sp = sp_ref[:]
    sp_o[: _NAT] = sp
    sp_o[_NAT:] = jnp.full((_ADD,), -1, jnp.int32)

    ta_o[: _NAT] = sp > 0
    ta_o[_NAT:] = jnp.zeros((_ADD,), jnp.bool_)

    bi_o[: _NAT] = bi_ref[:]
    bi_o[_NAT:] = jnp.full((_ADD,), _NSYS, jnp.int32)

    co_o[:, : _NAT] = co_ref[:, :]
    co_o[:, _NAT:] = jnp.zeros((3, _ADD), jnp.float32)

    na_o[: _NSYS] = na_ref[:]
    na_o[_NSYS:] = jnp.full((1,), _ADD, jnp.int32)

    ce_o[: _NSYS] = ce_ref[:]
    r = lax.broadcasted_iota(jnp.int32, (1, 3, 3), 1)
    c = lax.broadcasted_iota(jnp.int32, (1, 3, 3), 2)
    ce_o[_NSYS:] = jnp.where(r == c, 1.0, 0.0).astype(jnp.float32)

    ts_o[: _NSYS] = jnp.ones((_NSYS,), jnp.bool_)
    ts_o[_NSYS:] = jnp.zeros((1,), jnp.bool_)


def kernel(species, natoms, batch_index, coordinates, cells):
    coords_t = coordinates.T
    out = pl.pallas_call(
        _pad_kernel,
        out_shape=(
            jax.ShapeDtypeStruct((_PREV,), jnp.int32),
            jax.ShapeDtypeStruct((_NSYS + 1,), jnp.int32),
            jax.ShapeDtypeStruct((_PREV,), jnp.int32),
            jax.ShapeDtypeStruct((3, _PREV), jnp.float32),
            jax.ShapeDtypeStruct((_NSYS + 1, 3, 3), jnp.float32),
            jax.ShapeDtypeStruct((_PREV,), jnp.bool_),
            jax.ShapeDtypeStruct((_NSYS + 1,), jnp.bool_),
        ),
    )(species, natoms, batch_index, coords_t, cells)
    (species_out, natoms_out, batch_index_out,
     coords_out_t, cells_out, true_atoms, true_sys) = out
    coordinates_out = coords_out_t.T
    return (species_out, natoms_out, batch_index_out,
            coordinates_out, cells_out, true_atoms, true_sys)


# speedup vs baseline: 11.6488x; 1.5180x over previous
import jax
import jax.numpy as jnp
from jax import lax
from jax.experimental import pallas as pl

_NAT = 32768
_NSYS = 16
_PREV = int(1.2 * _NAT) + 1
_ADD = _PREV - _NAT


def _pad_kernel(sp_ref, na_ref, bi_ref, co_ref, ce_ref,
                sp_o, na_o, bi_o, co_o, ce_o, ta_o, ts_o):
    sp = sp_ref[:]
    sp_o[: _NAT] = sp
    sp_o[_NAT:] = jnp.full((_ADD,), -1, jnp.int32)

    ta_o[: _NAT] = sp > 0
    ta_o[_NAT:] = jnp.zeros((_ADD,), jnp.bool_)

    bi_o[: _NAT] = bi_ref[:]
    bi_o[_NAT:] = jnp.full((_ADD,), _NSYS, jnp.int32)

    co_o[:, : _NAT] = co_ref[:, :]
    co_o[:, _NAT:] = jnp.zeros((3, _ADD), jnp.float32)

    na_o[: _NSYS] = na_ref[:]
    na_o[_NSYS:] = jnp.full((1,), _ADD, jnp.int32)

    ce_o[:, :, : _NSYS] = ce_ref[:, :, :]
    r = lax.broadcasted_iota(jnp.int32, (3, 3, 1), 0)
    c = lax.broadcasted_iota(jnp.int32, (3, 3, 1), 1)
    ce_o[:, :, _NSYS:] = jnp.where(r == c, 1.0, 0.0).astype(jnp.float32)

    ts_o[: _NSYS] = jnp.ones((_NSYS,), jnp.bool_)
    ts_o[_NSYS:] = jnp.zeros((1,), jnp.bool_)


def kernel(species, natoms, batch_index, coordinates, cells):
    out = pl.pallas_call(
        _pad_kernel,
        out_shape=(
            jax.ShapeDtypeStruct((_PREV,), jnp.int32),
            jax.ShapeDtypeStruct((_NSYS + 1,), jnp.int32),
            jax.ShapeDtypeStruct((_PREV,), jnp.int32),
            jax.ShapeDtypeStruct((3, _PREV), jnp.float32),
            jax.ShapeDtypeStruct((3, 3, _NSYS + 1), jnp.float32),
            jax.ShapeDtypeStruct((_PREV,), jnp.bool_),
            jax.ShapeDtypeStruct((_NSYS + 1,), jnp.bool_),
        ),
    )(species, natoms, batch_index, coordinates.T, cells.transpose(1, 2, 0))
    (species_out, natoms_out, batch_index_out,
     coords_out_t, cells_out_t, true_atoms, true_sys) = out
    return (species_out, natoms_out, batch_index_out,
            coords_out_t.T, cells_out_t.transpose(2, 0, 1), true_atoms, true_sys)


# speedup vs baseline: 13.1262x; 1.1268x over previous
import jax
import jax.numpy as jnp
from jax import lax
from jax.experimental import pallas as pl

_NAT = 32768
_NSYS = 16
_PREV = int(1.2 * _NAT) + 1
_ADD = _PREV - _NAT


def _pad_kernel(sp_ref, na_ref, bi_ref, co_ref, ce_ref,
                sp_o, na_o, bi_o, co_o, ce_o, ta_o):
    sp = sp_ref[:]
    sp_o[: _NAT] = sp
    sp_o[_NAT:] = jnp.full((_ADD,), -1, jnp.int32)

    ta_o[: _NAT] = sp > 0
    ta_o[_NAT:] = jnp.zeros((_ADD,), jnp.bool_)

    bi_o[: _NAT] = bi_ref[:]
    bi_o[_NAT:] = jnp.full((_ADD,), _NSYS, jnp.int32)

    co_o[:, : _NAT] = co_ref[:, :]
    co_o[:, _NAT:] = jnp.zeros((3, _ADD), jnp.float32)

    na_o[: _NSYS] = na_ref[:]
    na_o[_NSYS:] = jnp.full((1,), _ADD, jnp.int32)

    ce_o[:, :, : _NSYS] = ce_ref[:, :, :]
    r = lax.broadcasted_iota(jnp.int32, (3, 3, 1), 0)
    c = lax.broadcasted_iota(jnp.int32, (3, 3, 1), 1)
    ce_o[:, :, _NSYS:] = jnp.where(r == c, 1.0, 0.0).astype(jnp.float32)


def kernel(species, natoms, batch_index, coordinates, cells):
    out = pl.pallas_call(
        _pad_kernel,
        out_shape=(
            jax.ShapeDtypeStruct((_PREV,), jnp.int32),
            jax.ShapeDtypeStruct((_NSYS + 1,), jnp.int32),
            jax.ShapeDtypeStruct((_PREV,), jnp.int32),
            jax.ShapeDtypeStruct((3, _PREV), jnp.float32),
            jax.ShapeDtypeStruct((3, 3, _NSYS + 1), jnp.float32),
            jax.ShapeDtypeStruct((_PREV,), jnp.bool_),
        ),
    )(species, natoms, batch_index, coordinates.T, cells.transpose(1, 2, 0))
    (species_out, natoms_out, batch_index_out,
     coords_out_t, cells_out_t, true_atoms) = out
    true_sys = jnp.arange(_NSYS + 1) < _NSYS
    return (species_out, natoms_out, batch_index_out,
            coords_out_t.T, cells_out_t.transpose(2, 0, 1), true_atoms, true_sys)


# speedup vs baseline: 13.2406x; 1.0087x over previous
import jax
import jax.numpy as jnp
from jax import lax
from jax.experimental import pallas as pl
from jax.experimental.pallas import tpu as pltpu

_NAT = 32768
_NSYS = 16
_PREV = int(1.2 * _NAT) + 1
_ADD = _PREV - _NAT

_HBM = pltpu.MemorySpace.HBM


def _pad_kernel(sp, na, bi, co, ce,
                spo, nao, bio, coo, ceo, tao,
                spv, biv, tav, cov, cev, nav, cei, nai, sems):
    cin = [
        pltpu.make_async_copy(sp, spv.at[pl.ds(0, _NAT)], sems.at[0]),
        pltpu.make_async_copy(bi, biv.at[pl.ds(0, _NAT)], sems.at[1]),
        pltpu.make_async_copy(co, cov.at[:, pl.ds(0, _NAT)], sems.at[2]),
        pltpu.make_async_copy(ce, cei, sems.at[3]),
        pltpu.make_async_copy(na, nai, sems.at[4]),
    ]
    for c in cin:
        c.start()

    cin[0].wait()
    spv[_NAT:] = jnp.full((_ADD,), -1, jnp.int32)
    tav[:] = (spv[:] > 0).astype(jnp.int8)
    c0 = pltpu.make_async_copy(spv, spo, sems.at[5])
    c0.start()
    c5 = pltpu.make_async_copy(tav, tao, sems.at[10])
    c5.start()

    cin[1].wait()
    biv[_NAT:] = jnp.full((_ADD,), _NSYS, jnp.int32)
    c1 = pltpu.make_async_copy(biv, bio, sems.at[6])
    c1.start()

    cin[2].wait()
    cov[:, _NAT:] = jnp.zeros((3, _ADD), jnp.float32)
    c2 = pltpu.make_async_copy(cov, coo, sems.at[7])
    c2.start()

    cin[3].wait()
    cev[:, :, : _NSYS] = cei[:, :, :]
    r = lax.broadcasted_iota(jnp.int32, (3, 3, 1), 0)
    c = lax.broadcasted_iota(jnp.int32, (3, 3, 1), 1)
    cev[:, :, _NSYS:] = jnp.where(r == c, 1.0, 0.0).astype(jnp.float32)
    c3 = pltpu.make_async_copy(cev, ceo, sems.at[8])
    c3.start()

    cin[4].wait()
    nav[: _NSYS] = nai[:]
    nav[_NSYS:] = jnp.full((1,), _ADD, jnp.int32)
    c4 = pltpu.make_async_copy(nav, nao, sems.at[9])
    c4.start()

    for cc in (c0, c1, c2, c3, c4, c5):
        cc.wait()


def kernel(species, natoms, batch_index, coordinates, cells):
    out = pl.pallas_call(
        _pad_kernel,
        out_shape=(
            jax.ShapeDtypeStruct((_PREV,), jnp.int32),
            jax.ShapeDtypeStruct((_NSYS + 1,), jnp.int32),
            jax.ShapeDtypeStruct((_PREV,), jnp.int32),
            jax.ShapeDtypeStruct((3, _PREV), jnp.float32),
            jax.ShapeDtypeStruct((3, 3, _NSYS + 1), jnp.float32),
            jax.ShapeDtypeStruct((_PREV,), jnp.int8),
        ),
        in_specs=[pl.BlockSpec(memory_space=_HBM)] * 5,
        out_specs=tuple([pl.BlockSpec(memory_space=_HBM)] * 6),
        scratch_shapes=[
            pltpu.VMEM((_PREV,), jnp.int32),
            pltpu.VMEM((_PREV,), jnp.int32),
            pltpu.VMEM((_PREV,), jnp.int8),
            pltpu.VMEM((3, _PREV), jnp.float32),
            pltpu.VMEM((3, 3, _NSYS + 1), jnp.float32),
            pltpu.VMEM((_NSYS + 1,), jnp.int32),
            pltpu.VMEM((3, 3, _NSYS), jnp.float32),
            pltpu.VMEM((_NSYS,), jnp.int32),
            pltpu.SemaphoreType.DMA((11,)),
        ],
    )(species, natoms, batch_index, coordinates.T, cells.transpose(1, 2, 0))
    (species_out, natoms_out, batch_index_out,
     coords_out_t, cells_out_t, ta_bytes) = out
    true_atoms = ta_bytes.view(jnp.bool_)
    true_sys = jnp.arange(_NSYS + 1) < _NSYS
    return (species_out, natoms_out, batch_index_out,
            coords_out_t.T, cells_out_t.transpose(2, 0, 1), true_atoms, true_sys)
